# baseline (device time: 51562 ns/iter reference)
import jax
import jax.numpy as jnp
from jax import lax
from jax.experimental import pallas as pl
from jax.experimental.pallas import tpu as pltpu

N_DEV = 16
M = 1024
N = 1024
CH = M // N_DEV
IH = 8
HC = N // 2

_GELU_C = 0.7978845608028654

_MESH = pl.DeviceIdType.MESH


def kernel(x, w_mat):
    def body(x_ref, w_ref, out_ref,
             sbL, sbR, pAL, pAR, pBL, pBR, sfL, sfR, fsL, fsR,
             agsL, agsR, mirL, mirR, agAL, agAR, agBL, agBR,
             p1_send, pA_recv, pB_recv, p2_send, fs_recv,
             mir_send, mir_recv, agA_send, agA_recv, agB_send, agB_recv):
        me = lax.axis_index("i")
        z = me // 4
        p = me % 4
        h = z // 2
        my_ih = (z % 2) * 4 + p
        mir_me = 4 * (3 - z) + p

        sb = (sbL, sbR)
        pA = (pAL, pAR)
        pB = (pBL, pBR)
        sf = (sfL, sfR)
        fs = (fsL, fsR)
        ags = (agsL, agsR)
        mirb = (mirL, mirR)
        agA = (agAL, agAR)
        agB = (agBL, agBR)
        cols = (slice(0, HC), slice(HC, N))

        def ih_to_idx(ihh):
            return 4 * (2 * h + ihh // 4) + ihh % 4

        def mirror_of(idx):
            return 4 * (3 - idx // 4) + idx % 4

        def partial(cidx):
            return jnp.dot(x_ref[pl.ds(cidx * CH, CH), :], w_ref[...],
                           preferred_element_type=jnp.float32)

        sends = []
        for off in range(IH - 1, 0, -1):
            peer_ih = (my_ih + off) % IH
            peer_idx = ih_to_idx(peer_ih)
            peer_mir = mirror_of(peer_idx)
            for j, cidx in ((0, peer_idx), (1, peer_mir)):
                part = partial(cidx)
                dsts = pA if j == 0 else pB
                for hf in (0, 1):
                    sb[hf][pl.ds(cidx, 1), :, :] = (
                        part[:, cols[hf]].astype(jnp.bfloat16)[None])
                    rsem = pA_recv if j == 0 else pB_recv
                    r = pltpu.make_async_remote_copy(
                        src_ref=sb[hf].at[cidx], dst_ref=dsts[hf].at[my_ih],
                        send_sem=p1_send.at[hf * 16 + 2 * off + j],
                        recv_sem=rsem.at[hf * IH + my_ih],
                        device_id=(peer_idx,), device_id_type=_MESH)
                    r.start()
                    sends.append(r)

        own = partial(me)
        ownm = partial(mir_me)
        for hf in (0, 1):
            pA[hf][pl.ds(my_ih, 1), :, :] = own[:, cols[hf]].astype(jnp.bfloat16)[None]
            pB[hf][pl.ds(my_ih, 1), :, :] = ownm[:, cols[hf]].astype(jnp.bfloat16)[None]

        for hf in (0, 1):
            for off in range(1, IH):
                src_ih = (my_ih + IH - off) % IH
                for buf, rsem in ((pA[hf], pA_recv), (pB[hf], pB_recv)):
                    pltpu.make_async_remote_copy(
                        src_ref=sf[hf], dst_ref=buf.at[src_ih],
                        send_sem=p2_send.at[hf],
                        recv_sem=rsem.at[hf * IH + src_ih],
                        device_id=(me,), device_id_type=_MESH).wait_recv()
            s_a = jnp.sum(pA[hf][...].astype(jnp.float32), axis=0)
            s_b = jnp.sum(pB[hf][...].astype(jnp.float32), axis=0)
            sf[hf][...] = s_b.astype(jnp.bfloat16)
            p2 = pltpu.make_async_remote_copy(
                src_ref=sf[hf], dst_ref=fs[hf],
                send_sem=p2_send.at[hf], recv_sem=fs_recv.at[hf],
                device_id=(mir_me,), device_id_type=_MESH)
            p2.start()
            sends.append(p2)
            pltpu.make_async_remote_copy(
                src_ref=sf[hf], dst_ref=fs[hf], send_sem=p2_send.at[hf],
                recv_sem=fs_recv.at[hf], device_id=(me,),
                device_id_type=_MESH).wait_recv()
            t = s_a + fs[hf][...].astype(jnp.float32)
            g = 0.5 * t * (1.0 + jnp.tanh(_GELU_C * (t + 0.044715 * t ** 3)))
            out_ref[pl.ds(me * CH, CH), cols[hf]] = g
            ags[hf][...] = g.astype(jnp.bfloat16)
            mr = pltpu.make_async_remote_copy(
                src_ref=ags[hf], dst_ref=mirb[hf],
                send_sem=mir_send.at[hf], recv_sem=mir_recv.at[hf],
                device_id=(mir_me,), device_id_type=_MESH)
            mr.start()
            sends.append(mr)
            for off in range(1, IH):
                peer_idx = ih_to_idx((my_ih + off) % IH)
                r = pltpu.make_async_remote_copy(
                    src_ref=ags[hf], dst_ref=agA[hf].at[my_ih],
                    send_sem=agA_send.at[hf * IH + off],
                    recv_sem=agA_recv.at[hf * IH + my_ih],
                    device_id=(peer_idx,), device_id_type=_MESH)
                r.start()
                sends.append(r)

        for hf in (0, 1):
            pltpu.make_async_remote_copy(
                src_ref=ags[hf], dst_ref=mirb[hf], send_sem=mir_send.at[hf],
                recv_sem=mir_recv.at[hf], device_id=(me,),
                device_id_type=_MESH).wait_recv()
            out_ref[pl.ds(mir_me * CH, CH), cols[hf]] = (
                mirb[hf][...].astype(jnp.float32))
            for off in range(1, IH):
                peer_idx = ih_to_idx((my_ih + off) % IH)
                r = pltpu.make_async_remote_copy(
                    src_ref=mirb[hf], dst_ref=agB[hf].at[my_ih],
                    send_sem=agB_send.at[hf * IH + off],
                    recv_sem=agB_recv.at[hf * IH + my_ih],
                    device_id=(peer_idx,), device_id_type=_MESH)
                r.start()
                sends.append(r)

        for hf in (0, 1):
            for off in range(1, IH):
                src_ih = (my_ih + IH - off) % IH
                src_idx = ih_to_idx(src_ih)
                pltpu.make_async_remote_copy(
                    src_ref=ags[hf], dst_ref=agA[hf].at[src_ih],
                    send_sem=agA_send.at[hf * IH],
                    recv_sem=agA_recv.at[hf * IH + src_ih],
                    device_id=(me,), device_id_type=_MESH).wait_recv()
                out_ref[pl.ds(src_idx * CH, CH), cols[hf]] = (
                    agA[hf][pl.ds(src_ih, 1), :, :][0].astype(jnp.float32))
            for off in range(1, IH):
                src_ih = (my_ih + IH - off) % IH
                far_idx = mirror_of(ih_to_idx(src_ih))
                pltpu.make_async_remote_copy(
                    src_ref=ags[hf], dst_ref=agB[hf].at[src_ih],
                    send_sem=agB_send.at[hf * IH],
                    recv_sem=agB_recv.at[hf * IH + src_ih],
                    device_id=(me,), device_id_type=_MESH).wait_recv()
                out_ref[pl.ds(far_idx * CH, CH), cols[hf]] = (
                    agB[hf][pl.ds(src_ih, 1), :, :][0].astype(jnp.float32))

        for r in sends:
            r.wait_send()

    bf = jnp.bfloat16
    return pl.pallas_call(
        body,
        out_shape=jax.ShapeDtypeStruct((M, N), jnp.float32),
        in_specs=[pl.BlockSpec(memory_space=pltpu.VMEM),
                  pl.BlockSpec(memory_space=pltpu.VMEM)],
        out_specs=pl.BlockSpec(memory_space=pltpu.VMEM),
        scratch_shapes=[
            pltpu.VMEM((N_DEV, CH, HC), bf),
            pltpu.VMEM((N_DEV, CH, HC), bf),
            pltpu.VMEM((IH, CH, HC), bf),
            pltpu.VMEM((IH, CH, HC), bf),
            pltpu.VMEM((IH, CH, HC), bf),
            pltpu.VMEM((IH, CH, HC), bf),
            pltpu.VMEM((CH, HC), bf),
            pltpu.VMEM((CH, HC), bf),
            pltpu.VMEM((CH, HC), bf),
            pltpu.VMEM((CH, HC), bf),
            pltpu.VMEM((CH, HC), bf),
            pltpu.VMEM((CH, HC), bf),
            pltpu.VMEM((CH, HC), bf),
            pltpu.VMEM((CH, HC), bf),
            pltpu.VMEM((IH, CH, HC), bf),
            pltpu.VMEM((IH, CH, HC), bf),
            pltpu.VMEM((IH, CH, HC), bf),
            pltpu.VMEM((IH, CH, HC), bf),
            pltpu.SemaphoreType.DMA((32,)),
            pltpu.SemaphoreType.DMA((16,)),
            pltpu.SemaphoreType.DMA((16,)),
            pltpu.SemaphoreType.DMA((2,)),
            pltpu.SemaphoreType.DMA((2,)),
            pltpu.SemaphoreType.DMA((2,)),
            pltpu.SemaphoreType.DMA((2,)),
            pltpu.SemaphoreType.DMA((16,)),
            pltpu.SemaphoreType.DMA((16,)),
            pltpu.SemaphoreType.DMA((16,)),
            pltpu.SemaphoreType.DMA((16,)),
        ],
    )(x, w_mat)


# device time: 50495 ns/iter; 1.0211x vs baseline; 1.0211x over previous
import jax
import jax.numpy as jnp
from jax import lax
from jax.experimental import pallas as pl
from jax.experimental.pallas import tpu as pltpu

N_DEV = 16
M = 1024
N = 1024
CH = M // N_DEV
IH = 8

_GELU_C = 0.7978845608028654
_MESH = pl.DeviceIdType.MESH

_P1_ORDER = (4, 5, 3, 6, 2, 7, 1)


def kernel(x, w_mat):
    def body(x_ref, w_ref, out_ref, part_ref, spb, pAB, sf, fs, ag_src,
             mir_buf, agA, agB,
             p1_send, pAB_recv, p2_send, fs_recv,
             mir_send, mir_recv, agA_send, agA_recv, agB_send, agB_recv):
        me = lax.axis_index("i")
        z = me // 4
        p = me % 4
        h = z // 2
        my_ih = (z % 2) * 4 + p
        mir_me = 4 * (3 - z) + p

        def ih_to_idx(ihh):
            return 4 * (2 * h + ihh // 4) + ihh % 4

        def mirror_of(idx):
            return 4 * (3 - idx // 4) + idx % 4

        part_ref[...] = jnp.dot(x_ref[...], w_ref[...],
                                preferred_element_type=jnp.float32)

        def rows(cidx):
            return part_ref[pl.ds(cidx * CH, CH), :]

        sends = []
        for off in _P1_ORDER:
            peer_ih = (my_ih + off) % IH
            peer_idx = ih_to_idx(peer_ih)
            peer_mir = mirror_of(peer_idx)
            spb[off - 1, 0:CH, :] = rows(peer_idx).astype(jnp.bfloat16)
            spb[off - 1, CH:2 * CH, :] = rows(peer_mir).astype(jnp.bfloat16)
            r = pltpu.make_async_remote_copy(
                src_ref=spb.at[off - 1], dst_ref=pAB.at[my_ih],
                send_sem=p1_send.at[off], recv_sem=pAB_recv.at[my_ih],
                device_id=(peer_idx,), device_id_type=_MESH)
            r.start()
            sends.append(r)

        pAB[pl.ds(my_ih, 1), 0:CH, :] = rows(me).astype(jnp.bfloat16)[None]
        pAB[pl.ds(my_ih, 1), CH:2 * CH, :] = rows(mir_me).astype(jnp.bfloat16)[None]

        for off in range(1, IH):
            src_ih = (my_ih + IH - off) % IH
            pltpu.make_async_remote_copy(
                src_ref=spb.at[0], dst_ref=pAB.at[src_ih],
                send_sem=p1_send.at[0], recv_sem=pAB_recv.at[src_ih],
                device_id=(me,), device_id_type=_MESH).wait_recv()

        s_ab = jnp.sum(pAB[...].astype(jnp.float32), axis=0)
        s_a = s_ab[0:CH, :]
        sf[...] = s_ab[CH:2 * CH, :].astype(jnp.bfloat16)

        p2 = pltpu.make_async_remote_copy(
            src_ref=sf, dst_ref=fs,
            send_sem=p2_send.at[0], recv_sem=fs_recv.at[0],
            device_id=(mir_me,), device_id_type=_MESH)
        p2.start()
        sends.append(p2)
        pltpu.make_async_remote_copy(
            src_ref=sf, dst_ref=fs, send_sem=p2_send.at[0],
            recv_sem=fs_recv.at[0], device_id=(me,),
            device_id_type=_MESH).wait_recv()

        t = s_a + fs[...].astype(jnp.float32)
        g = 0.5 * t * (1.0 + jnp.tanh(_GELU_C * (t + 0.044715 * t ** 3)))
        out_ref[pl.ds(me * CH, CH), :] = g
        ag_src[...] = g.astype(jnp.bfloat16)

        mr = pltpu.make_async_remote_copy(
            src_ref=ag_src, dst_ref=mir_buf,
            send_sem=mir_send.at[0], recv_sem=mir_recv.at[0],
            device_id=(mir_me,), device_id_type=_MESH)
        mr.start()
        sends.append(mr)
        for off in _P1_ORDER:
            peer_idx = ih_to_idx((my_ih + off) % IH)
            r = pltpu.make_async_remote_copy(
                src_ref=ag_src, dst_ref=agA.at[my_ih],
                send_sem=agA_send.at[off], recv_sem=agA_recv.at[my_ih],
                device_id=(peer_idx,), device_id_type=_MESH)
            r.start()
            sends.append(r)

        pltpu.make_async_remote_copy(
            src_ref=ag_src, dst_ref=mir_buf, send_sem=mir_send.at[0],
            recv_sem=mir_recv.at[0], device_id=(me,),
            device_id_type=_MESH).wait_recv()
        out_ref[pl.ds(mir_me * CH, CH), :] = mir_buf[...].astype(jnp.float32)
        for off in _P1_ORDER:
            peer_idx = ih_to_idx((my_ih + off) % IH)
            r = pltpu.make_async_remote_copy(
                src_ref=mir_buf, dst_ref=agB.at[my_ih],
                send_sem=agB_send.at[off], recv_sem=agB_recv.at[my_ih],
                device_id=(peer_idx,), device_id_type=_MESH)
            r.start()
            sends.append(r)

        for off in range(1, IH):
            src_ih = (my_ih + IH - off) % IH
            src_idx = ih_to_idx(src_ih)
            pltpu.make_async_remote_copy(
                src_ref=ag_src, dst_ref=agA.at[src_ih],
                send_sem=agA_send.at[0], recv_sem=agA_recv.at[src_ih],
                device_id=(me,), device_id_type=_MESH).wait_recv()
            out_ref[pl.ds(src_idx * CH, CH), :] = (
                agA[pl.ds(src_ih, 1), :, :][0].astype(jnp.float32))
        for off in range(1, IH):
            src_ih = (my_ih + IH - off) % IH
            far_idx = mirror_of(ih_to_idx(src_ih))
            pltpu.make_async_remote_copy(
                src_ref=ag_src, dst_ref=agB.at[src_ih],
                send_sem=agB_send.at[0], recv_sem=agB_recv.at[src_ih],
                device_id=(me,), device_id_type=_MESH).wait_recv()
            out_ref[pl.ds(far_idx * CH, CH), :] = (
                agB[pl.ds(src_ih, 1), :, :][0].astype(jnp.float32))

        for r in sends:
            r.wait_send()

    bf = jnp.bfloat16
    return pl.pallas_call(
        body,
        out_shape=jax.ShapeDtypeStruct((M, N), jnp.float32),
        in_specs=[pl.BlockSpec(memory_space=pltpu.VMEM),
                  pl.BlockSpec(memory_space=pltpu.VMEM)],
        out_specs=pl.BlockSpec(memory_space=pltpu.VMEM),
        scratch_shapes=[
            pltpu.VMEM((M, N), jnp.float32),
            pltpu.VMEM((IH - 1, 2 * CH, N), bf),
            pltpu.VMEM((IH, 2 * CH, N), bf),
            pltpu.VMEM((CH, N), bf),
            pltpu.VMEM((CH, N), bf),
            pltpu.VMEM((CH, N), bf),
            pltpu.VMEM((CH, N), bf),
            pltpu.VMEM((IH, CH, N), bf),
            pltpu.VMEM((IH, CH, N), bf),
            pltpu.SemaphoreType.DMA((IH,)),
            pltpu.SemaphoreType.DMA((IH,)),
            pltpu.SemaphoreType.DMA((1,)),
            pltpu.SemaphoreType.DMA((1,)),
            pltpu.SemaphoreType.DMA((1,)),
            pltpu.SemaphoreType.DMA((1,)),
            pltpu.SemaphoreType.DMA((IH,)),
            pltpu.SemaphoreType.DMA((IH,)),
            pltpu.SemaphoreType.DMA((IH,)),
            pltpu.SemaphoreType.DMA((IH,)),
        ],
    )(x, w_mat)


# device time: 44708 ns/iter; 1.1533x vs baseline; 1.1294x over previous
import jax
import jax.numpy as jnp
from jax import lax
from jax.experimental import pallas as pl
from jax.experimental.pallas import tpu as pltpu

N_DEV = 16
M = 1024
N = 1024
CH = M // N_DEV
IH = 8

_GELU_C = 0.7978845608028654
_MESH = pl.DeviceIdType.MESH

_P1_ORDER = (4, 5, 3, 6, 2, 7, 1)


def kernel(x, w_mat):
    def body(x_ref, w_ref, out_ref, send_buf, pA, pB, sf, fs, ag_src,
             mir_buf, agA, agB,
             p1a_send, p1b_send, pA_recv, pB_recv,
             p2_send, fs_recv, mir_send, mir_recv,
             agA_send, agA_recv, agB_send, agB_recv):
        me = lax.axis_index("i")
        z = me // 4
        p = me % 4
        h = z // 2
        my_ih = (z % 2) * 4 + p
        mir_me = 4 * (3 - z) + p

        def ih_to_idx(ihh):
            return 4 * (2 * h + ihh // 4) + ihh % 4

        def mirror_of(idx):
            return 4 * (3 - idx // 4) + idx % 4

        barrier_sem = pltpu.get_barrier_semaphore()
        for off in range(1, IH):
            pl.semaphore_signal(barrier_sem, inc=1,
                                device_id=(ih_to_idx((my_ih + off) % IH),),
                                device_id_type=_MESH)
        pl.semaphore_signal(barrier_sem, inc=1, device_id=(mir_me,),
                            device_id_type=_MESH)
        pl.semaphore_wait(barrier_sem, IH)

        sends = []
        for off in _P1_ORDER:
            peer_ih = (my_ih + off) % IH
            peer_idx = ih_to_idx(peer_ih)
            peer_mir = mirror_of(peer_idx)
            for j, cidx in ((1, peer_mir), (0, peer_idx)):
                part = jnp.dot(x_ref[pl.ds(cidx * CH, CH), :], w_ref[...],
                               preferred_element_type=jnp.float32)
                send_buf[pl.ds(cidx, 1), :, :] = (
                    part.astype(jnp.bfloat16)[None])
                buf, rsem, ssem = ((pA, pA_recv, p1a_send) if j == 0
                                   else (pB, pB_recv, p1b_send))
                r = pltpu.make_async_remote_copy(
                    src_ref=send_buf.at[cidx], dst_ref=buf.at[my_ih],
                    send_sem=ssem.at[off], recv_sem=rsem.at[my_ih],
                    device_id=(peer_idx,), device_id_type=_MESH)
                r.start()
                sends.append(r)

        own_b = jnp.dot(x_ref[pl.ds(mir_me * CH, CH), :], w_ref[...],
                        preferred_element_type=jnp.float32)
        own_a = jnp.dot(x_ref[pl.ds(me * CH, CH), :], w_ref[...],
                        preferred_element_type=jnp.float32)

        s_b = own_b
        for off in range(1, IH):
            src_ih = (my_ih + IH - off) % IH
            pltpu.make_async_remote_copy(
                src_ref=send_buf.at[0], dst_ref=pB.at[src_ih],
                send_sem=p1b_send.at[0], recv_sem=pB_recv.at[src_ih],
                device_id=(me,), device_id_type=_MESH).wait_recv()
            s_b = s_b + pB[pl.ds(src_ih, 1), :, :][0].astype(jnp.float32)
        sf[...] = s_b.astype(jnp.bfloat16)

        p2 = pltpu.make_async_remote_copy(
            src_ref=sf, dst_ref=fs,
            send_sem=p2_send.at[0], recv_sem=fs_recv.at[0],
            device_id=(mir_me,), device_id_type=_MESH)
        p2.start()
        sends.append(p2)

        s_a = own_a
        for off in range(1, IH):
            src_ih = (my_ih + IH - off) % IH
            pltpu.make_async_remote_copy(
                src_ref=send_buf.at[0], dst_ref=pA.at[src_ih],
                send_sem=p1a_send.at[0], recv_sem=pA_recv.at[src_ih],
                device_id=(me,), device_id_type=_MESH).wait_recv()
            s_a = s_a + pA[pl.ds(src_ih, 1), :, :][0].astype(jnp.float32)

        pltpu.make_async_remote_copy(
            src_ref=sf, dst_ref=fs, send_sem=p2_send.at[0],
            recv_sem=fs_recv.at[0], device_id=(me,),
            device_id_type=_MESH).wait_recv()

        t = s_a + fs[...].astype(jnp.float32)
        g = 0.5 * t * (1.0 + jnp.tanh(_GELU_C * (t + 0.044715 * t ** 3)))
        ag_src[...] = g.astype(jnp.bfloat16)

        mr = pltpu.make_async_remote_copy(
            src_ref=ag_src, dst_ref=mir_buf,
            send_sem=mir_send.at[0], recv_sem=mir_recv.at[0],
            device_id=(mir_me,), device_id_type=_MESH)
        mr.start()
        sends.append(mr)
        for off in _P1_ORDER:
            peer_idx = ih_to_idx((my_ih + off) % IH)
            r = pltpu.make_async_remote_copy(
                src_ref=ag_src, dst_ref=agA.at[my_ih],
                send_sem=agA_send.at[off], recv_sem=agA_recv.at[my_ih],
                device_id=(peer_idx,), device_id_type=_MESH)
            r.start()
            sends.append(r)
        out_ref[pl.ds(me * CH, CH), :] = g

        pltpu.make_async_remote_copy(
            src_ref=ag_src, dst_ref=mir_buf, send_sem=mir_send.at[0],
            recv_sem=mir_recv.at[0], device_id=(me,),
            device_id_type=_MESH).wait_recv()
        for off in _P1_ORDER:
            peer_idx = ih_to_idx((my_ih + off) % IH)
            r = pltpu.make_async_remote_copy(
                src_ref=mir_buf, dst_ref=agB.at[my_ih],
                send_sem=agB_send.at[off], recv_sem=agB_recv.at[my_ih],
                device_id=(peer_idx,), device_id_type=_MESH)
            r.start()
            sends.append(r)
        out_ref[pl.ds(mir_me * CH, CH), :] = mir_buf[...].astype(jnp.float32)

        for off in range(1, IH):
            src_ih = (my_ih + IH - off) % IH
            src_idx = ih_to_idx(src_ih)
            pltpu.make_async_remote_copy(
                src_ref=ag_src, dst_ref=agA.at[src_ih],
                send_sem=agA_send.at[0], recv_sem=agA_recv.at[src_ih],
                device_id=(me,), device_id_type=_MESH).wait_recv()
            out_ref[pl.ds(src_idx * CH, CH), :] = (
                agA[pl.ds(src_ih, 1), :, :][0].astype(jnp.float32))
        for off in range(1, IH):
            src_ih = (my_ih + IH - off) % IH
            far_idx = mirror_of(ih_to_idx(src_ih))
            pltpu.make_async_remote_copy(
                src_ref=ag_src, dst_ref=agB.at[src_ih],
                send_sem=agB_send.at[0], recv_sem=agB_recv.at[src_ih],
                device_id=(me,), device_id_type=_MESH).wait_recv()
            out_ref[pl.ds(far_idx * CH, CH), :] = (
                agB[pl.ds(src_ih, 1), :, :][0].astype(jnp.float32))

        for r in sends:
            r.wait_send()

    bf = jnp.bfloat16
    return pl.pallas_call(
        body,
        out_shape=jax.ShapeDtypeStruct((M, N), jnp.float32),
        in_specs=[pl.BlockSpec(memory_space=pltpu.VMEM),
                  pl.BlockSpec(memory_space=pltpu.VMEM)],
        out_specs=pl.BlockSpec(memory_space=pltpu.VMEM),
        scratch_shapes=[
            pltpu.VMEM((N_DEV, CH, N), bf),
            pltpu.VMEM((IH, CH, N), bf),
            pltpu.VMEM((IH, CH, N), bf),
            pltpu.VMEM((CH, N), bf),
            pltpu.VMEM((CH, N), bf),
            pltpu.VMEM((CH, N), bf),
            pltpu.VMEM((CH, N), bf),
            pltpu.VMEM((IH, CH, N), bf),
            pltpu.VMEM((IH, CH, N), bf),
            pltpu.SemaphoreType.DMA((IH,)),
            pltpu.SemaphoreType.DMA((IH,)),
            pltpu.SemaphoreType.DMA((IH,)),
            pltpu.SemaphoreType.DMA((IH,)),
            pltpu.SemaphoreType.DMA((1,)),
            pltpu.SemaphoreType.DMA((1,)),
            pltpu.SemaphoreType.DMA((1,)),
            pltpu.SemaphoreType.DMA((1,)),
            pltpu.SemaphoreType.DMA((IH,)),
            pltpu.SemaphoreType.DMA((IH,)),
            pltpu.SemaphoreType.DMA((IH,)),
            pltpu.SemaphoreType.DMA((IH,)),
        ],
        compiler_params=pltpu.CompilerParams(collective_id=0),
    )(x, w_mat)
